# baseline (speedup 1.0000x reference)
import functools

import jax
import jax.numpy as jnp
from jax import lax
from jax.experimental import pallas as pl
from jax.experimental.pallas import tpu as pltpu
from jax.experimental.pallas import tpu_sc as plsc

N_IDX = 100000
D = 128
C = 128
NW = 32
CPW = 25
N_CHUNKS = NW * CPW
B_PAD = N_CHUNKS * C

_mesh = plsc.VectorSubcoreMesh(core_axis_name="c", subcore_axis_name="s")


@functools.partial(
    pl.kernel,
    mesh=_mesh,
    out_type=jax.ShapeDtypeStruct((B_PAD, D), jnp.float32),
    scratch_types=[
        pltpu.VMEM((CPW, C), jnp.int32),
        pltpu.VMEM((2, C, D), jnp.float32),
        pltpu.SemaphoreType.DMA,
        pltpu.SemaphoreType.DMA,
    ],
)
def _sc_gather(x_hbm, idx_hbm, out_hbm, idx_v, rows_v, gsem, osem):
    w = lax.axis_index("s") * 2 + lax.axis_index("c")
    chunk0 = w * CPW

    def start_gather(j, buf):
        pltpu.async_copy(x_hbm.at[idx_v.at[j]], rows_v.at[buf], gsem)

    def wait_gather(buf):
        pltpu.make_async_copy(x_hbm.at[pl.ds(0, C)], rows_v.at[buf], gsem).wait()

    def start_out(j, buf):
        pltpu.async_copy(rows_v.at[buf], out_hbm.at[pl.ds((chunk0 + j) * C, C)],
                         osem)

    def wait_out():
        pltpu.make_async_copy(rows_v.at[0], out_hbm.at[pl.ds(0, C)], osem).wait()

    pltpu.sync_copy(idx_hbm.at[w], idx_v)
    start_gather(0, 0)

    def body(g, carry):
        for k, buf in ((0, 0), (1, 1)):
            j = 2 * g + k

            @pl.when(j >= 1)
            def _():
                wait_out()
            start_gather(j + 1, 1 - buf)
            wait_gather(buf)
            start_out(j, buf)
        return carry

    lax.fori_loop(0, (CPW - 1) // 2, body, 0)

    wait_out()
    wait_gather(0)
    start_out(CPW - 1, 0)
    wait_out()


def kernel(x, idx):
    idx32 = idx.astype(jnp.int32)
    idx_pad = jnp.zeros((B_PAD,), jnp.int32).at[:N_IDX].set(idx32)
    out = _sc_gather(x, idx_pad.reshape(NW, CPW, C))
    return out[:N_IDX]
